# baseline (speedup 1.0000x reference)
import functools

import jax
import jax.numpy as jnp
import numpy as np
from jax.experimental import pallas as pl
from jax.experimental.pallas import tpu as pltpu

LANE = 128

_LAYERS = [
    (1, 32, 4, 4, 4, False),
    (1, 32, 4, 4, 4, True),
    (2, 16, 4, 8, 8, False),
    (2, 16, 8, 8, 8, True),
    (4, 8, 8, 16, 16, False),
    (4, 8, 16, 16, 16, False),
    (4, 8, 16, 16, 16, True),
    (8, 4, 16, 32, 32, False),
    (8, 4, 32, 32, 32, False),
    (8, 4, 32, 32, 32, True),
    (16, 2, 32, 64, 32, False),
    (16, 2, 32, 32, 32, False),
    (16, 2, 32, 32, 32, True),
]

_SIDX = {1: 1, 2: 2, 4: 3, 8: 4, 16: 5}


def _band_mask(Wi, s_in, Co):
    kw = np.arange(3)[:, None, None]
    x = (np.arange(LANE) // s_in)[None, :, None]
    w = (np.arange(LANE) // Co)[None, None, :]
    d = (x == w + kw - 1).astype(np.float32)
    d[:, Wi * s_in:, :] = 0.0
    d[:, :, Wi * Co:] = 0.0
    return d


_DMASKS = [_band_mask(Wi, s_in, Co) for (_s, Wi, _Ci, s_in, Co, _p) in _LAYERS]

_E = np.zeros((96, LANE), np.float32)
for _c in range(3):
    _E[_c * 32 + np.arange(32), np.arange(32) * 4 + _c] = 1.0


def _keep_masks(R):
    h = np.arange(R) % 32
    m = np.ones((R, 8), np.float32)
    m[h == 0, 0] = 0.0
    for s, j in _SIDX.items():
        m[h == 32 - s, j] = 0.0
    return m


def _fused_kernel(x_ref, e_ref, w_ref, f_ref, b_ref, m_ref, p_ref, o_ref, *, nb):
    R = nb * 32

    xcat = jnp.concatenate([x_ref[:, c, :, :].reshape(R, 32) for c in range(3)],
                           axis=1)
    x = jnp.dot(xcat, e_ref[...],
                preferred_element_type=jnp.float32).astype(jnp.bfloat16)

    k0 = m_ref[:, 0:1]
    for l, (s, _Wi, _Ci, _si, Co, pool) in enumerate(_LAYERS):
        xp = jnp.pad(x, ((s, s), (0, 0)))
        k2 = m_ref[:, _SIDX[s]:_SIDX[s] + 1]
        x3 = jnp.concatenate([xp[0:R] * k0, xp[s:s + R],
                              xp[2 * s:2 * s + R] * k2], axis=1)
        acc = jnp.dot(x3, w_ref[l], preferred_element_type=jnp.float32)
        y = jnp.maximum(acc + b_ref[l:l + 1, :], 0.0)
        if pool:
            ysh = jnp.pad(y[s:], ((0, s), (0, 0)))
            y = jnp.maximum(y, ysh)
            ysw = jnp.pad(y[:, Co:], ((0, 0), (0, Co)))
            y = jnp.maximum(y, ysw)
        x = y.astype(jnp.bfloat16)

    a = jnp.dot(p_ref[...], x,
                preferred_element_type=jnp.float32).astype(jnp.bfloat16)
    h = jnp.dot(a, f_ref[0], preferred_element_type=jnp.float32)
    h = jnp.maximum(h + b_ref[13:14, :], 0.0).astype(jnp.bfloat16)
    h = jnp.dot(h, f_ref[1], preferred_element_type=jnp.float32)
    o_ref[...] = jnp.maximum(h + b_ref[14:15, :], 0.0)


def kernel(x_nchw, conv_w_0, conv_b_0, conv_w_1, conv_b_1, conv_w_2, conv_b_2,
           conv_w_3, conv_b_3, conv_w_4, conv_b_4, conv_w_5, conv_b_5,
           conv_w_6, conv_b_6, conv_w_7, conv_b_7, conv_w_8, conv_b_8,
           conv_w_9, conv_b_9, conv_w_10, conv_b_10, conv_w_11, conv_b_11,
           conv_w_12, conv_b_12, fc_w_0, fc_b_0, fc_w_1, fc_b_1):
    conv_w = [conv_w_0, conv_w_1, conv_w_2, conv_w_3, conv_w_4, conv_w_5,
              conv_w_6, conv_w_7, conv_w_8, conv_w_9, conv_w_10, conv_w_11,
              conv_w_12]
    conv_b = [conv_b_0, conv_b_1, conv_b_2, conv_b_3, conv_b_4, conv_b_5,
              conv_b_6, conv_b_7, conv_b_8, conv_b_9, conv_b_10, conv_b_11,
              conv_b_12]

    N = x_nchw.shape[0]
    nb = min(128, N)
    assert N % nb == 0
    R = nb * 32

    x_bf = x_nchw.astype(jnp.bfloat16)

    bs, biases = [], []
    for l, (_s, Wi, Ci, s_in, Co, _p) in enumerate(_LAYERS):
        wl = conv_w[l][:, :, :Ci, :Co]
        wl = jnp.pad(wl, ((0, 0), (0, 0), (0, s_in - Ci), (0, 0)))
        wt = jnp.broadcast_to(wl[:, :, None, :, None, :],
                              (3, 3, Wi, s_in, Wi, Co))
        wt = wt.reshape(3, 3, Wi * s_in, Wi * Co)
        wt = jnp.pad(wt, ((0, 0), (0, 0), (0, LANE - Wi * s_in),
                          (0, LANE - Wi * Co)))
        b3 = (wt * jnp.asarray(_DMASKS[l], wt.dtype)).sum(axis=1)
        bs.append(b3.reshape(3 * LANE, LANE))
        bl = jnp.broadcast_to(conv_b[l][:Co], (Wi, Co)).reshape(Wi * Co)
        biases.append(jnp.pad(bl, (0, LANE - Wi * Co)).astype(jnp.float32))
    w_all = jnp.stack(bs)
    f_all = jnp.stack([fc_w_0.astype(jnp.bfloat16),
                       fc_w_1.astype(jnp.bfloat16)])
    b_all = jnp.stack(
        biases + [fc_b_0.astype(jnp.float32), fc_b_1.astype(jnp.float32),
                  jnp.zeros((LANE,), jnp.float32)])

    e_mat = jnp.asarray(_E, jnp.bfloat16)
    m_mat = jnp.asarray(_keep_masks(R), jnp.bfloat16)
    p_np = np.zeros((nb, R), np.float32)
    p_np[np.arange(nb), np.arange(nb) * 32] = 1.0
    p_mat = jnp.asarray(p_np, jnp.bfloat16)

    flops_per_block = (2 * R * 96 * LANE
                       + 13 * 2 * R * 384 * LANE
                       + 2 * nb * R * LANE
                       + 2 * 2 * nb * LANE * LANE)
    flops = (N // nb) * flops_per_block
    bytes_accessed = x_bf.size * 2 + w_all.size * 2 + b_all.size * 4 + N * LANE * 4

    return pl.pallas_call(
        functools.partial(_fused_kernel, nb=nb),
        out_shape=jax.ShapeDtypeStruct((N, LANE), jnp.float32),
        grid=(N // nb,),
        in_specs=[
            pl.BlockSpec((nb, 3, 32, 32), lambda n: (n, 0, 0, 0)),
            pl.BlockSpec((96, LANE), lambda n: (0, 0)),
            pl.BlockSpec((13, 3 * LANE, LANE), lambda n: (0, 0, 0)),
            pl.BlockSpec((2, LANE, LANE), lambda n: (0, 0, 0)),
            pl.BlockSpec((16, LANE), lambda n: (0, 0)),
            pl.BlockSpec((R, 8), lambda n: (0, 0)),
            pl.BlockSpec((nb, R), lambda n: (0, 0)),
        ],
        out_specs=pl.BlockSpec((nb, LANE), lambda n: (n, 0)),
        compiler_params=pltpu.CompilerParams(
            dimension_semantics=("parallel",),
            vmem_limit_bytes=48 * 1024 * 1024),
        cost_estimate=pl.CostEstimate(flops=int(flops), transcendentals=0,
                                      bytes_accessed=int(bytes_accessed)),
    )(x_bf, e_mat, w_all, f_all, b_all, m_mat, p_mat)


# speedup vs baseline: 1.2200x; 1.2200x over previous
import functools

import jax
import jax.numpy as jnp
import numpy as np
from jax.experimental import pallas as pl
from jax.experimental.pallas import tpu as pltpu

LANE = 128

_LAYERS = [
    (1, 32, 4, 4, 4, False),
    (1, 32, 4, 4, 4, True),
    (2, 16, 4, 8, 8, False),
    (2, 16, 8, 8, 8, True),
    (4, 8, 8, 16, 16, False),
    (4, 8, 16, 16, 16, False),
    (4, 8, 16, 16, 16, True),
    (8, 4, 16, 32, 32, False),
    (8, 4, 32, 32, 32, False),
    (8, 4, 32, 32, 32, True),
    (16, 2, 32, 64, 32, False),
    (16, 2, 32, 32, 32, False),
    (16, 2, 32, 32, 32, True),
]

_SIDX = {1: 1, 2: 2, 4: 3, 8: 4, 16: 5}

_CSTRIDE = {7: 1, 8: 1, 9: 1, 10: 2, 11: 2, 12: 2}
_CSIDX = {7: 1, 8: 1, 9: 1, 10: 2, 11: 2, 12: 2}


def _band_mask(Wi, s_in, Co):
    kw = np.arange(3)[:, None, None]
    x = (np.arange(LANE) // s_in)[None, :, None]
    w = (np.arange(LANE) // Co)[None, None, :]
    d = (x == w + kw - 1).astype(np.float32)
    d[:, Wi * s_in:, :] = 0.0
    d[:, :, Wi * Co:] = 0.0
    return d


_DMASKS = [_band_mask(Wi, s_in, Co) for (_s, Wi, _Ci, s_in, Co, _p) in _LAYERS]

_E = np.zeros((96, LANE), np.float32)
for _c in range(3):
    _E[_c * 32 + np.arange(32), np.arange(32) * 4 + _c] = 1.0


def _keep_masks(R):
    h = np.arange(R) % 32
    m = np.ones((R, 8), np.float32)
    m[h == 0, 0] = 0.0
    for s, j in _SIDX.items():
        m[h == 32 - s, j] = 0.0
    return m


def _keep_masks2(R8):
    j = np.arange(R8) % 4
    m = np.ones((R8, 8), np.float32)
    m[j == 0, 0] = 0.0
    m[j == 3, 1] = 0.0
    m[j == 2, 2] = 0.0
    return m


def _fused_kernel(x_ref, e_ref, w_ref, f_ref, b_ref, m_ref, m2_ref, p_ref,
                  o_ref, *, nb):
    R = nb * 32

    xcat = jnp.concatenate([x_ref[:, c, :, :].reshape(R, 32) for c in range(3)],
                           axis=1)
    x = jnp.dot(xcat, e_ref[...],
                preferred_element_type=jnp.float32).astype(jnp.bfloat16)

    for l, (s, _Wi, _Ci, _si, Co, pool) in enumerate(_LAYERS):
        if l == 7:
            x = x.reshape(R // 8, 8, LANE)[:, 0, :]
        Rl = x.shape[0]
        sl = s if l < 7 else _CSTRIDE[l]
        mref = m_ref if l < 7 else m2_ref
        j2 = _SIDX[s] if l < 7 else _CSIDX[l]
        k0 = mref[:, 0:1]
        k2 = mref[:, j2:j2 + 1]
        xp = jnp.pad(x, ((sl, sl), (0, 0)))
        x3 = jnp.concatenate([xp[0:Rl] * k0, xp[sl:sl + Rl],
                              xp[2 * sl:2 * sl + Rl] * k2], axis=1)
        acc = jnp.dot(x3, w_ref[l], preferred_element_type=jnp.float32)
        y = jnp.maximum(acc + b_ref[l:l + 1, :], 0.0)
        if pool:
            ysh = jnp.pad(y[sl:], ((0, sl), (0, 0)))
            y = jnp.maximum(y, ysh)
            ysw = jnp.pad(y[:, Co:], ((0, 0), (0, Co)))
            y = jnp.maximum(y, ysw)
        x = y.astype(jnp.bfloat16)

    a = jnp.dot(p_ref[...], x,
                preferred_element_type=jnp.float32).astype(jnp.bfloat16)
    h = jnp.dot(a, f_ref[0], preferred_element_type=jnp.float32)
    h = jnp.maximum(h + b_ref[13:14, :], 0.0).astype(jnp.bfloat16)
    h = jnp.dot(h, f_ref[1], preferred_element_type=jnp.float32)
    o_ref[...] = jnp.maximum(h + b_ref[14:15, :], 0.0)


def kernel(x_nchw, conv_w_0, conv_b_0, conv_w_1, conv_b_1, conv_w_2, conv_b_2,
           conv_w_3, conv_b_3, conv_w_4, conv_b_4, conv_w_5, conv_b_5,
           conv_w_6, conv_b_6, conv_w_7, conv_b_7, conv_w_8, conv_b_8,
           conv_w_9, conv_b_9, conv_w_10, conv_b_10, conv_w_11, conv_b_11,
           conv_w_12, conv_b_12, fc_w_0, fc_b_0, fc_w_1, fc_b_1):
    conv_w = [conv_w_0, conv_w_1, conv_w_2, conv_w_3, conv_w_4, conv_w_5,
              conv_w_6, conv_w_7, conv_w_8, conv_w_9, conv_w_10, conv_w_11,
              conv_w_12]
    conv_b = [conv_b_0, conv_b_1, conv_b_2, conv_b_3, conv_b_4, conv_b_5,
              conv_b_6, conv_b_7, conv_b_8, conv_b_9, conv_b_10, conv_b_11,
              conv_b_12]

    N = x_nchw.shape[0]
    nb = min(128, N)
    assert N % nb == 0
    R = nb * 32

    x_bf = x_nchw.astype(jnp.bfloat16)

    bs, biases = [], []
    for l, (_s, Wi, Ci, s_in, Co, _p) in enumerate(_LAYERS):
        wl = conv_w[l][:, :, :Ci, :Co]
        wl = jnp.pad(wl, ((0, 0), (0, 0), (0, s_in - Ci), (0, 0)))
        wt = jnp.broadcast_to(wl[:, :, None, :, None, :],
                              (3, 3, Wi, s_in, Wi, Co))
        wt = wt.reshape(3, 3, Wi * s_in, Wi * Co)
        wt = jnp.pad(wt, ((0, 0), (0, 0), (0, LANE - Wi * s_in),
                          (0, LANE - Wi * Co)))
        b3 = (wt * jnp.asarray(_DMASKS[l], wt.dtype)).sum(axis=1)
        bs.append(b3.reshape(3 * LANE, LANE))
        bl = jnp.broadcast_to(conv_b[l][:Co], (Wi, Co)).reshape(Wi * Co)
        biases.append(jnp.pad(bl, (0, LANE - Wi * Co)).astype(jnp.float32))
    w_all = jnp.stack(bs)
    f_all = jnp.stack([fc_w_0.astype(jnp.bfloat16),
                       fc_w_1.astype(jnp.bfloat16)])
    b_all = jnp.stack(
        biases + [fc_b_0.astype(jnp.float32), fc_b_1.astype(jnp.float32),
                  jnp.zeros((LANE,), jnp.float32)])

    e_mat = jnp.asarray(_E, jnp.bfloat16)
    m_mat = jnp.asarray(_keep_masks(R), jnp.bfloat16)
    m2_mat = jnp.asarray(_keep_masks2(R // 8), jnp.bfloat16)
    p_np = np.zeros((nb, R // 8), np.float32)
    p_np[np.arange(nb), np.arange(nb) * 4] = 1.0
    p_mat = jnp.asarray(p_np, jnp.bfloat16)

    flops_per_block = (2 * R * 96 * LANE
                       + 13 * 2 * R * 384 * LANE
                       + 2 * nb * R * LANE
                       + 2 * 2 * nb * LANE * LANE)
    flops = (N // nb) * flops_per_block
    bytes_accessed = x_bf.size * 2 + w_all.size * 2 + b_all.size * 4 + N * LANE * 4

    return pl.pallas_call(
        functools.partial(_fused_kernel, nb=nb),
        out_shape=jax.ShapeDtypeStruct((N, LANE), jnp.float32),
        grid=(N // nb,),
        in_specs=[
            pl.BlockSpec((nb, 3, 32, 32), lambda n: (n, 0, 0, 0)),
            pl.BlockSpec((96, LANE), lambda n: (0, 0)),
            pl.BlockSpec((13, 3 * LANE, LANE), lambda n: (0, 0, 0)),
            pl.BlockSpec((2, LANE, LANE), lambda n: (0, 0, 0)),
            pl.BlockSpec((16, LANE), lambda n: (0, 0)),
            pl.BlockSpec((R, 8), lambda n: (0, 0)),
            pl.BlockSpec((R // 8, 8), lambda n: (0, 0)),
            pl.BlockSpec((nb, R // 8), lambda n: (0, 0)),
        ],
        out_specs=pl.BlockSpec((nb, LANE), lambda n: (n, 0)),
        compiler_params=pltpu.CompilerParams(
            dimension_semantics=("parallel",),
            vmem_limit_bytes=48 * 1024 * 1024),
        cost_estimate=pl.CostEstimate(flops=int(flops), transcendentals=0,
                                      bytes_accessed=int(bytes_accessed)),
    )(x_bf, e_mat, w_all, f_all, b_all, m_mat, m2_mat, p_mat)


# speedup vs baseline: 1.3146x; 1.0776x over previous
import functools

import jax
import jax.numpy as jnp
import numpy as np
from jax.experimental import pallas as pl
from jax.experimental.pallas import tpu as pltpu

LANE = 128

_LAYERS = [
    (1, 32, 4, 4, 4, False),
    (1, 32, 4, 4, 4, True),
    (2, 16, 4, 8, 8, False),
    (2, 16, 8, 8, 8, True),
    (4, 8, 8, 16, 16, False),
    (4, 8, 16, 16, 16, False),
    (4, 8, 16, 16, 16, True),
    (8, 4, 16, 32, 32, False),
    (8, 4, 32, 32, 32, False),
    (8, 4, 32, 32, 32, True),
    (16, 2, 32, 64, 32, False),
    (16, 2, 32, 32, 32, False),
    (16, 2, 32, 32, 32, True),
]

_SIDX = {1: 1, 2: 2, 4: 3, 8: 4, 16: 5}

_CSTRIDE = {7: 1, 8: 1, 9: 1, 10: 2, 11: 2, 12: 2}
_CSIDX = {7: 1, 8: 1, 9: 1, 10: 2, 11: 2, 12: 2}


def _prep_constants():
    nl = len(_LAYERS)
    u = np.zeros((nl, 32, LANE), np.float32)
    v = np.zeros((nl, 32, LANE), np.float32)
    d = np.zeros((nl, 3, LANE, LANE), np.float32)
    for l, (_s, Wi, _Ci, s_in, Co, _p) in enumerate(_LAYERS):
        p = np.arange(Wi * s_in)
        i = p % s_in
        sel = i < 32
        u[l, i[sel], p[sel]] = 1.0
        q = np.arange(Wi * Co)
        v[l, q % Co, q] = 1.0
        kw = np.arange(3)[:, None, None]
        x = (np.arange(LANE) // s_in)[None, :, None]
        w = (np.arange(LANE) // Co)[None, None, :]
        d[l] = (x == w + kw - 1).astype(np.float32)
    return u, v, d


_U, _V, _D = _prep_constants()

_E = np.zeros((96, LANE), np.float32)
for _c in range(3):
    _E[_c * 32 + np.arange(32), np.arange(32) * 4 + _c] = 1.0


def _keep_masks(R):
    h = np.arange(R) % 32
    m = np.ones((R, 8), np.float32)
    m[h == 0, 0] = 0.0
    for s, j in _SIDX.items():
        m[h == 32 - s, j] = 0.0
    return m


def _keep_masks2(R8):
    j = np.arange(R8) % 4
    m = np.ones((R8, 8), np.float32)
    m[j == 0, 0] = 0.0
    m[j == 3, 1] = 0.0
    m[j == 2, 2] = 0.0
    return m


def _fused_kernel(x_ref, e_ref, w_ref, f_ref, b_ref, m_ref, m2_ref, p_ref,
                  o_ref, *, nb):
    R = nb * 32

    xcat = jnp.concatenate([x_ref[:, c, :, :].reshape(R, 32) for c in range(3)],
                           axis=1)
    x = jnp.dot(xcat, e_ref[...],
                preferred_element_type=jnp.float32).astype(jnp.bfloat16)

    for l, (s, _Wi, _Ci, _si, Co, pool) in enumerate(_LAYERS):
        if l == 7:
            x = x.reshape(R // 8, 8, LANE)[:, 0, :]
        Rl = x.shape[0]
        sl = s if l < 7 else _CSTRIDE[l]
        mref = m_ref if l < 7 else m2_ref
        j2 = _SIDX[s] if l < 7 else _CSIDX[l]
        k0 = mref[:, 0:1]
        k2 = mref[:, j2:j2 + 1]
        xp = jnp.pad(x, ((sl, sl), (0, 0)))
        x3 = jnp.concatenate([xp[0:Rl] * k0, xp[sl:sl + Rl],
                              xp[2 * sl:2 * sl + Rl] * k2], axis=1)
        acc = jnp.dot(x3, w_ref[l], preferred_element_type=jnp.float32)
        y = jnp.maximum(acc + b_ref[l:l + 1, :], 0.0)
        if pool:
            ysh = jnp.pad(y[sl:], ((0, sl), (0, 0)))
            y = jnp.maximum(y, ysh)
            ysw = jnp.pad(y[:, Co:], ((0, 0), (0, Co)))
            y = jnp.maximum(y, ysw)
        x = y.astype(jnp.bfloat16)

    a = jnp.dot(p_ref[...], x,
                preferred_element_type=jnp.float32).astype(jnp.bfloat16)
    h = jnp.dot(a, f_ref[0], preferred_element_type=jnp.float32)
    h = jnp.maximum(h + b_ref[13:14, :], 0.0).astype(jnp.bfloat16)
    h = jnp.dot(h, f_ref[1], preferred_element_type=jnp.float32)
    o_ref[...] = jnp.maximum(h + b_ref[14:15, :], 0.0)


def kernel(x_nchw, conv_w_0, conv_b_0, conv_w_1, conv_b_1, conv_w_2, conv_b_2,
           conv_w_3, conv_b_3, conv_w_4, conv_b_4, conv_w_5, conv_b_5,
           conv_w_6, conv_b_6, conv_w_7, conv_b_7, conv_w_8, conv_b_8,
           conv_w_9, conv_b_9, conv_w_10, conv_b_10, conv_w_11, conv_b_11,
           conv_w_12, conv_b_12, fc_w_0, fc_b_0, fc_w_1, fc_b_1):
    conv_w = [conv_w_0, conv_w_1, conv_w_2, conv_w_3, conv_w_4, conv_w_5,
              conv_w_6, conv_w_7, conv_w_8, conv_w_9, conv_w_10, conv_w_11,
              conv_w_12]
    conv_b = [conv_b_0, conv_b_1, conv_b_2, conv_b_3, conv_b_4, conv_b_5,
              conv_b_6, conv_b_7, conv_b_8, conv_b_9, conv_b_10, conv_b_11,
              conv_b_12]

    N = x_nchw.shape[0]
    nb = min(128, N)
    assert N % nb == 0
    R = nb * 32

    x_bf = x_nchw.astype(jnp.bfloat16)

    wp = jnp.stack([w[:, :, :32, :32] for w in conv_w]).astype(jnp.float32)
    u = jnp.asarray(_U)
    v = jnp.asarray(_V)
    t = jnp.einsum("lhkio,loq->lhkiq", wp, v)
    t = jnp.einsum("lhkiq,lip->lhkpq", t, u)
    b3 = (t * jnp.asarray(_D)[:, None]).sum(axis=2)
    w_all = b3.reshape(13, 3 * LANE, LANE).astype(jnp.bfloat16)
    f_all = jnp.stack([fc_w_0.astype(jnp.bfloat16),
                       fc_w_1.astype(jnp.bfloat16)])
    bp = jnp.stack(conv_b)[:, :32].astype(jnp.float32)
    bias13 = jnp.einsum("lo,loq->lq", bp, v)
    b_all = jnp.concatenate(
        [bias13, fc_b_0[None].astype(jnp.float32),
         fc_b_1[None].astype(jnp.float32),
         jnp.zeros((1, LANE), jnp.float32)], axis=0)

    e_mat = jnp.asarray(_E, jnp.bfloat16)
    m_mat = jnp.asarray(_keep_masks(R), jnp.bfloat16)
    m2_mat = jnp.asarray(_keep_masks2(R // 8), jnp.bfloat16)
    p_np = np.zeros((nb, R // 8), np.float32)
    p_np[np.arange(nb), np.arange(nb) * 4] = 1.0
    p_mat = jnp.asarray(p_np, jnp.bfloat16)

    flops_per_block = (2 * R * 96 * LANE
                       + 13 * 2 * R * 384 * LANE
                       + 2 * nb * R * LANE
                       + 2 * 2 * nb * LANE * LANE)
    flops = (N // nb) * flops_per_block
    bytes_accessed = x_bf.size * 2 + w_all.size * 2 + b_all.size * 4 + N * LANE * 4

    return pl.pallas_call(
        functools.partial(_fused_kernel, nb=nb),
        out_shape=jax.ShapeDtypeStruct((N, LANE), jnp.float32),
        grid=(N // nb,),
        in_specs=[
            pl.BlockSpec((nb, 3, 32, 32), lambda n: (n, 0, 0, 0)),
            pl.BlockSpec((96, LANE), lambda n: (0, 0)),
            pl.BlockSpec((13, 3 * LANE, LANE), lambda n: (0, 0, 0)),
            pl.BlockSpec((2, LANE, LANE), lambda n: (0, 0, 0)),
            pl.BlockSpec((16, LANE), lambda n: (0, 0)),
            pl.BlockSpec((R, 8), lambda n: (0, 0)),
            pl.BlockSpec((R // 8, 8), lambda n: (0, 0)),
            pl.BlockSpec((nb, R // 8), lambda n: (0, 0)),
        ],
        out_specs=pl.BlockSpec((nb, LANE), lambda n: (n, 0)),
        compiler_params=pltpu.CompilerParams(
            dimension_semantics=("parallel",),
            vmem_limit_bytes=48 * 1024 * 1024),
        cost_estimate=pl.CostEstimate(flops=int(flops), transcendentals=0,
                                      bytes_accessed=int(bytes_accessed)),
    )(x_bf, e_mat, w_all, f_all, b_all, m_mat, m2_mat, p_mat)


# speedup vs baseline: 1.3347x; 1.0153x over previous
import functools

import jax
import jax.numpy as jnp
import numpy as np
from jax.experimental import pallas as pl
from jax.experimental.pallas import tpu as pltpu

LANE = 128

_LAYERS = [
    (1, 32, 4, 4, 4, False),
    (1, 32, 4, 4, 4, True),
    (2, 16, 4, 8, 8, False),
    (2, 16, 8, 8, 8, True),
    (4, 8, 8, 16, 16, False),
    (4, 8, 16, 16, 16, False),
    (4, 8, 16, 16, 16, True),
    (8, 4, 16, 32, 32, False),
    (8, 4, 32, 32, 32, False),
    (8, 4, 32, 32, 32, True),
    (16, 2, 32, 64, 32, False),
    (16, 2, 32, 32, 32, False),
    (16, 2, 32, 32, 32, True),
]

_SIDX = {1: 1, 2: 2, 4: 3, 8: 4, 16: 5}

_CSTRIDE = {7: 1, 8: 1, 9: 1, 10: 2, 11: 2, 12: 2}
_CSIDX = {7: 1, 8: 1, 9: 1, 10: 2, 11: 2, 12: 2}


def _prep_constants():
    nl = len(_LAYERS)
    u = np.zeros((nl, 32, LANE), np.float32)
    v = np.zeros((nl, 32, LANE), np.float32)
    d = np.zeros((nl, 3, LANE, LANE), np.float32)
    for l, (_s, Wi, _Ci, s_in, Co, _p) in enumerate(_LAYERS):
        p = np.arange(Wi * s_in)
        i = p % s_in
        sel = i < 32
        u[l, i[sel], p[sel]] = 1.0
        q = np.arange(Wi * Co)
        v[l, q % Co, q] = 1.0
        kw = np.arange(3)[:, None, None]
        x = (np.arange(LANE) // s_in)[None, :, None]
        w = (np.arange(LANE) // Co)[None, None, :]
        d[l] = (x == w + kw - 1).astype(np.float32)
    return u, v, d


_U, _V, _D = _prep_constants()

_E = np.zeros((96, LANE), np.float32)
for _c in range(3):
    _E[_c * 32 + np.arange(32), np.arange(32) * 4 + _c] = 1.0


def _keep_masks(R):
    h = np.arange(R) % 32
    m = np.ones((R, 8), np.float32)
    m[h == 0, 0] = 0.0
    for s, j in _SIDX.items():
        m[h == 32 - s, j] = 0.0
    return m


def _keep_masks2(R8):
    j = np.arange(R8) % 4
    m = np.ones((R8, 8), np.float32)
    m[j == 0, 0] = 0.0
    m[j == 3, 1] = 0.0
    m[j == 2, 2] = 0.0
    return m


def _fused_kernel(x_ref, e_ref, w_ref, f_ref, b_ref, m_ref, m2_ref, p_ref,
                  o_ref, *, nb):
    R = nb * 32

    xcat = jnp.concatenate([x_ref[:, c, :, :].reshape(R, 32) for c in range(3)],
                           axis=1)
    x = jnp.dot(xcat, e_ref[...],
                preferred_element_type=jnp.float32).astype(jnp.bfloat16)

    for l, (s, _Wi, _Ci, _si, Co, pool) in enumerate(_LAYERS):
        if l == 7:
            x = x.reshape(R // 8, 8, LANE)[:, 0, :]
        Rl = x.shape[0]
        sl = s if l < 7 else _CSTRIDE[l]
        mref = m_ref if l < 7 else m2_ref
        j2 = _SIDX[s] if l < 7 else _CSIDX[l]
        k0 = mref[:, 0:1]
        k2 = mref[:, j2:j2 + 1]
        xp = jnp.pad(x, ((sl, sl), (0, 0)))
        x3 = jnp.concatenate([xp[0:Rl] * k0, xp[sl:sl + Rl],
                              xp[2 * sl:2 * sl + Rl] * k2], axis=1)
        acc = jnp.dot(x3, w_ref[l], preferred_element_type=jnp.float32)
        y = jnp.maximum(acc + b_ref[l:l + 1, :], 0.0)
        if pool:
            ysh = jnp.pad(y[sl:], ((0, sl), (0, 0)))
            y = jnp.maximum(y, ysh)
            ysw = jnp.pad(y[:, Co:], ((0, 0), (0, Co)))
            y = jnp.maximum(y, ysw)
        x = y.astype(jnp.bfloat16)

    a = jnp.dot(p_ref[...], x,
                preferred_element_type=jnp.float32).astype(jnp.bfloat16)
    h = jnp.dot(a, f_ref[0], preferred_element_type=jnp.float32)
    h = jnp.maximum(h + b_ref[13:14, :], 0.0).astype(jnp.bfloat16)
    h = jnp.dot(h, f_ref[1], preferred_element_type=jnp.float32)
    o_ref[...] = jnp.maximum(h + b_ref[14:15, :], 0.0)


def kernel(x_nchw, conv_w_0, conv_b_0, conv_w_1, conv_b_1, conv_w_2, conv_b_2,
           conv_w_3, conv_b_3, conv_w_4, conv_b_4, conv_w_5, conv_b_5,
           conv_w_6, conv_b_6, conv_w_7, conv_b_7, conv_w_8, conv_b_8,
           conv_w_9, conv_b_9, conv_w_10, conv_b_10, conv_w_11, conv_b_11,
           conv_w_12, conv_b_12, fc_w_0, fc_b_0, fc_w_1, fc_b_1):
    conv_w = [conv_w_0, conv_w_1, conv_w_2, conv_w_3, conv_w_4, conv_w_5,
              conv_w_6, conv_w_7, conv_w_8, conv_w_9, conv_w_10, conv_w_11,
              conv_w_12]
    conv_b = [conv_b_0, conv_b_1, conv_b_2, conv_b_3, conv_b_4, conv_b_5,
              conv_b_6, conv_b_7, conv_b_8, conv_b_9, conv_b_10, conv_b_11,
              conv_b_12]

    N = x_nchw.shape[0]
    nb = min(128, N)
    assert N % nb == 0
    R = nb * 32

    x_bf = x_nchw.astype(jnp.bfloat16)

    wp = jnp.stack([w[:, :, :32, :32] for w in conv_w]).astype(jnp.float32)
    u = jnp.asarray(_U)
    v = jnp.asarray(_V)
    hi = jax.lax.Precision.HIGHEST
    t = jnp.einsum("lhkio,loq->lhkiq", wp, v, precision=hi)
    t = jnp.einsum("lhkiq,lip->lhkpq", t, u, precision=hi)
    b3 = (t * jnp.asarray(_D)[:, None]).sum(axis=2)
    w_all = b3.reshape(13, 3 * LANE, LANE).astype(jnp.bfloat16)
    f_all = jnp.stack([fc_w_0.astype(jnp.bfloat16),
                       fc_w_1.astype(jnp.bfloat16)])
    bp = jnp.stack(conv_b)[:, :32].astype(jnp.float32)
    bias13 = jnp.einsum("lo,loq->lq", bp, v, precision=hi)
    b_all = jnp.concatenate(
        [bias13, fc_b_0[None].astype(jnp.float32),
         fc_b_1[None].astype(jnp.float32),
         jnp.zeros((1, LANE), jnp.float32)], axis=0)

    e_mat = jnp.asarray(_E, jnp.bfloat16)
    m_mat = jnp.asarray(_keep_masks(R), jnp.bfloat16)
    m2_mat = jnp.asarray(_keep_masks2(R // 8), jnp.bfloat16)
    p_np = np.zeros((nb, R // 8), np.float32)
    p_np[np.arange(nb), np.arange(nb) * 4] = 1.0
    p_mat = jnp.asarray(p_np, jnp.bfloat16)

    flops_per_block = (2 * R * 96 * LANE
                       + 13 * 2 * R * 384 * LANE
                       + 2 * nb * R * LANE
                       + 2 * 2 * nb * LANE * LANE)
    flops = (N // nb) * flops_per_block
    bytes_accessed = x_bf.size * 2 + w_all.size * 2 + b_all.size * 4 + N * LANE * 4

    return pl.pallas_call(
        functools.partial(_fused_kernel, nb=nb),
        out_shape=jax.ShapeDtypeStruct((N, LANE), jnp.float32),
        grid=(N // nb,),
        in_specs=[
            pl.BlockSpec((nb, 3, 32, 32), lambda n: (n, 0, 0, 0)),
            pl.BlockSpec((96, LANE), lambda n: (0, 0)),
            pl.BlockSpec((13, 3 * LANE, LANE), lambda n: (0, 0, 0)),
            pl.BlockSpec((2, LANE, LANE), lambda n: (0, 0, 0)),
            pl.BlockSpec((16, LANE), lambda n: (0, 0)),
            pl.BlockSpec((R, 8), lambda n: (0, 0)),
            pl.BlockSpec((R // 8, 8), lambda n: (0, 0)),
            pl.BlockSpec((nb, R // 8), lambda n: (0, 0)),
        ],
        out_specs=pl.BlockSpec((nb, LANE), lambda n: (n, 0)),
        compiler_params=pltpu.CompilerParams(
            dimension_semantics=("parallel",),
            vmem_limit_bytes=48 * 1024 * 1024),
        cost_estimate=pl.CostEstimate(flops=int(flops), transcendentals=0,
                                      bytes_accessed=int(bytes_accessed)),
    )(x_bf, e_mat, w_all, f_all, b_all, m_mat, m2_mat, p_mat)


# speedup vs baseline: 1.3752x; 1.0304x over previous
import functools

import jax
import jax.numpy as jnp
import numpy as np
from jax.experimental import pallas as pl
from jax.experimental.pallas import tpu as pltpu

LANE = 128

_LAYERS = [
    (1, 32, 4, 4, 4, False),
    (1, 32, 4, 4, 4, True),
    (2, 16, 4, 8, 8, False),
    (2, 16, 8, 8, 8, True),
    (4, 8, 8, 16, 16, False),
    (4, 8, 16, 16, 16, False),
    (4, 8, 16, 16, 16, True),
    (8, 4, 16, 32, 32, False),
    (8, 4, 32, 32, 32, False),
    (8, 4, 32, 32, 32, True),
    (16, 2, 32, 64, 32, False),
    (16, 2, 32, 32, 32, False),
    (16, 2, 32, 32, 32, True),
]

_SIDX = {1: 1, 2: 2, 4: 3, 8: 4, 16: 5}

_CSTRIDE = {7: 1, 8: 1, 9: 1, 10: 2, 11: 2, 12: 2}
_CSIDX = {7: 1, 8: 1, 9: 1, 10: 2, 11: 2, 12: 2}


def _prep_constants():
    nl = len(_LAYERS)
    u = np.zeros((nl, 32, LANE), np.float32)
    v = np.zeros((nl, 32, LANE), np.float32)
    d = np.zeros((nl, 3, LANE, LANE), np.float32)
    for l, (_s, Wi, _Ci, s_in, Co, _p) in enumerate(_LAYERS):
        p = np.arange(Wi * s_in)
        i = p % s_in
        sel = i < 32
        u[l, i[sel], p[sel]] = 1.0
        q = np.arange(Wi * Co)
        v[l, q % Co, q] = 1.0
        kw = np.arange(3)[:, None, None]
        x = (np.arange(LANE) // s_in)[None, :, None]
        w = (np.arange(LANE) // Co)[None, None, :]
        d[l] = (x == w + kw - 1).astype(np.float32)
    return u, v, d


_U, _V, _D = _prep_constants()

_E = np.zeros((96, LANE), np.float32)
for _c in range(3):
    _E[_c * 32 + np.arange(32), np.arange(32) * 4 + _c] = 1.0


def _keep_masks(R):
    h = np.arange(R) % 32
    m = np.ones((R, 8), np.float32)
    m[h == 0, 0] = 0.0
    for s, j in _SIDX.items():
        m[h == 32 - s, j] = 0.0
    return m


def _keep_masks2(R8):
    j = np.arange(R8) % 4
    m = np.ones((R8, 8), np.float32)
    m[j == 0, 0] = 0.0
    m[j == 3, 1] = 0.0
    m[j == 2, 2] = 0.0
    return m


def _fused_kernel(x_ref, e_ref, w_ref, f_ref, b_ref, m_ref, m2_ref, p_ref,
                  o_ref, *, nb):
    R = nb * 32

    xcat = jnp.concatenate([x_ref[:, c, :, :].reshape(R, 32) for c in range(3)],
                           axis=1)
    x = jnp.dot(xcat, e_ref[...],
                preferred_element_type=jnp.float32).astype(jnp.bfloat16)

    for l, (s, _Wi, _Ci, _si, Co, pool) in enumerate(_LAYERS):
        if l == 7:
            x = x.reshape(R // 8, 8, LANE)[:, 0, :]
        Rl = x.shape[0]
        sl = s if l < 7 else _CSTRIDE[l]
        mref = m_ref if l < 7 else m2_ref
        j2 = _SIDX[s] if l < 7 else _CSIDX[l]
        k0 = mref[:, 0:1]
        k2 = mref[:, j2:j2 + 1]
        xp = jnp.pad(x, ((sl, sl), (0, 0)))
        x3 = jnp.concatenate([xp[0:Rl] * k0, xp[sl:sl + Rl],
                              xp[2 * sl:2 * sl + Rl] * k2], axis=1)
        acc = jnp.dot(x3, w_ref[l], preferred_element_type=jnp.float32)
        y = jnp.maximum(acc + b_ref[l:l + 1, :], 0.0)
        if pool:
            ysh = jnp.pad(y[sl:], ((0, sl), (0, 0)))
            y = jnp.maximum(y, ysh)
            ysw = jnp.pad(y[:, Co:], ((0, 0), (0, Co)))
            y = jnp.maximum(y, ysw)
        x = y.astype(jnp.bfloat16)

    a = jnp.dot(p_ref[...], x,
                preferred_element_type=jnp.float32).astype(jnp.bfloat16)
    h = jnp.dot(a, f_ref[0], preferred_element_type=jnp.float32)
    h = jnp.maximum(h + b_ref[13:14, :], 0.0).astype(jnp.bfloat16)
    h = jnp.dot(h, f_ref[1], preferred_element_type=jnp.float32)
    o_ref[...] = jnp.maximum(h + b_ref[14:15, :], 0.0)


def kernel(x_nchw, conv_w_0, conv_b_0, conv_w_1, conv_b_1, conv_w_2, conv_b_2,
           conv_w_3, conv_b_3, conv_w_4, conv_b_4, conv_w_5, conv_b_5,
           conv_w_6, conv_b_6, conv_w_7, conv_b_7, conv_w_8, conv_b_8,
           conv_w_9, conv_b_9, conv_w_10, conv_b_10, conv_w_11, conv_b_11,
           conv_w_12, conv_b_12, fc_w_0, fc_b_0, fc_w_1, fc_b_1):
    conv_w = [conv_w_0, conv_w_1, conv_w_2, conv_w_3, conv_w_4, conv_w_5,
              conv_w_6, conv_w_7, conv_w_8, conv_w_9, conv_w_10, conv_w_11,
              conv_w_12]
    conv_b = [conv_b_0, conv_b_1, conv_b_2, conv_b_3, conv_b_4, conv_b_5,
              conv_b_6, conv_b_7, conv_b_8, conv_b_9, conv_b_10, conv_b_11,
              conv_b_12]

    N = x_nchw.shape[0]
    nb = min(256, N)
    assert N % nb == 0
    R = nb * 32

    x_bf = x_nchw.astype(jnp.bfloat16)

    wp = jnp.stack([w[:, :, :32, :32] for w in conv_w]).astype(jnp.float32)
    u = jnp.asarray(_U)
    v = jnp.asarray(_V)
    hi = jax.lax.Precision.HIGHEST
    t = jnp.einsum("lhkio,loq->lhkiq", wp, v, precision=hi)
    t = jnp.einsum("lhkiq,lip->lhkpq", t, u, precision=hi)
    b3 = (t * jnp.asarray(_D)[:, None]).sum(axis=2)
    w_all = b3.reshape(13, 3 * LANE, LANE).astype(jnp.bfloat16)
    f_all = jnp.stack([fc_w_0.astype(jnp.bfloat16),
                       fc_w_1.astype(jnp.bfloat16)])
    bp = jnp.stack(conv_b)[:, :32].astype(jnp.float32)
    bias13 = jnp.einsum("lo,loq->lq", bp, v, precision=hi)
    b_all = jnp.concatenate(
        [bias13, fc_b_0[None].astype(jnp.float32),
         fc_b_1[None].astype(jnp.float32),
         jnp.zeros((1, LANE), jnp.float32)], axis=0)

    e_mat = jnp.asarray(_E, jnp.bfloat16)
    m_mat = jnp.asarray(_keep_masks(R), jnp.bfloat16)
    m2_mat = jnp.asarray(_keep_masks2(R // 8), jnp.bfloat16)
    p_np = np.zeros((nb, R // 8), np.float32)
    p_np[np.arange(nb), np.arange(nb) * 4] = 1.0
    p_mat = jnp.asarray(p_np, jnp.bfloat16)

    flops_per_block = (2 * R * 96 * LANE
                       + 13 * 2 * R * 384 * LANE
                       + 2 * nb * R * LANE
                       + 2 * 2 * nb * LANE * LANE)
    flops = (N // nb) * flops_per_block
    bytes_accessed = x_bf.size * 2 + w_all.size * 2 + b_all.size * 4 + N * LANE * 4

    return pl.pallas_call(
        functools.partial(_fused_kernel, nb=nb),
        out_shape=jax.ShapeDtypeStruct((N, LANE), jnp.float32),
        grid=(N // nb,),
        in_specs=[
            pl.BlockSpec((nb, 3, 32, 32), lambda n: (n, 0, 0, 0)),
            pl.BlockSpec((96, LANE), lambda n: (0, 0)),
            pl.BlockSpec((13, 3 * LANE, LANE), lambda n: (0, 0, 0)),
            pl.BlockSpec((2, LANE, LANE), lambda n: (0, 0, 0)),
            pl.BlockSpec((16, LANE), lambda n: (0, 0)),
            pl.BlockSpec((R, 8), lambda n: (0, 0)),
            pl.BlockSpec((R // 8, 8), lambda n: (0, 0)),
            pl.BlockSpec((nb, R // 8), lambda n: (0, 0)),
        ],
        out_specs=pl.BlockSpec((nb, LANE), lambda n: (n, 0)),
        compiler_params=pltpu.CompilerParams(
            dimension_semantics=("parallel",),
            vmem_limit_bytes=48 * 1024 * 1024),
        cost_estimate=pl.CostEstimate(flops=int(flops), transcendentals=0,
                                      bytes_accessed=int(bytes_accessed)),
    )(x_bf, e_mat, w_all, f_all, b_all, m_mat, m2_mat, p_mat)
